# baseline (speedup 1.0000x reference)
import jax
import jax.numpy as jnp
from jax import lax
from jax.experimental import pallas as pl
from jax.experimental.pallas import tpu as pltpu
from jax.experimental.pallas import tpu_sc as plsc

N_NODES = 10000
D = 128
NC = 2
NS = 16
NW = NC * NS
K = 128
B = 8
AROWS = 10112
RPT = AROWS // NS
CROWS = 10240
HR = CROWS // D


def _make_agg(nch: int, with_cnt: bool):
    assert nch % B == 0
    nblk = nch // B
    rpt_pad = nch + B
    mesh = plsc.VectorSubcoreMesh(core_axis_name="c", subcore_axis_name="s")
    out_type = [jax.ShapeDtypeStruct((NC * AROWS, D), jnp.float32)]
    scratch = [
        pltpu.VMEM((B, K), jnp.int32),
        pltpu.VMEM((B, K), jnp.int32),
        pltpu.VMEM((K, D), jnp.float32),
        pltpu.VMEM((K, D), jnp.float32),
        pltpu.VMEM_SHARED((AROWS, D), jnp.float32),
        pltpu.SemaphoreType.DMA,
        pltpu.SemaphoreType.DMA,
    ]
    if with_cnt:
        out_type.append(jax.ShapeDtypeStruct((NC * HR, D), jnp.float32))
        scratch += [
            pltpu.VMEM((HR, D), jnp.float32),
            pltpu.VMEM((HR,), jnp.int32),
            pltpu.VMEM_SHARED((HR, D), jnp.float32),
        ]

    def body(feat, srcm, dstm, zacc, *rest):
        if with_cnt:
            (riota, out, cnt_out, srcb, dstb, rows0, rows1, acc,
             gsem, ssem, hist_v, riota_v, cacc) = rest
        else:
            (out, srcb, dstb, rows0, rows1, acc, gsem, ssem) = rest
        cid = lax.axis_index("c")
        sid = lax.axis_index("s")
        wid = cid * NS + sid
        base = wid * rpt_pad

        pltpu.sync_copy(zacc, acc.at[pl.ds(sid * RPT, RPT)])
        if with_cnt:
            pltpu.sync_copy(zacc.at[pl.ds(0, HR)], hist_v)
            pltpu.sync_copy(riota, riota_v)

            @pl.when(sid < HR // 8)
            def _():
                pltpu.sync_copy(zacc.at[pl.ds(0, 8)],
                                cacc.at[pl.ds(sid * 8, 8)])

        plsc.subcore_barrier()

        ones16 = jnp.ones((16,), jnp.float32)
        bufs = (rows0, rows1)

        def load_blk(blk):
            pltpu.sync_copy(srcm.at[pl.ds(base + blk * B, B)], srcb)
            pltpu.sync_copy(dstm.at[pl.ds(base + blk * B, B)], dstb)

        def blk_body(blk, first):
            for c in range(B):
                rb = bufs[c % 2]
                rn = bufs[(c + 1) % 2]
                pltpu.make_async_copy(feat.at[srcb.at[c]], rb, gsem).wait()
                pltpu.async_copy(rb, acc.at[dstb.at[c]], ssem, add=True)
                if not (first and c == 0):
                    pltpu.make_async_copy(rn, acc.at[dstb.at[c]], ssem).wait()
                if c < B - 1:
                    pltpu.async_copy(feat.at[srcb.at[c + 1]], rn, gsem)
                if with_cnt:
                    for t in range(K // 16):
                        idx = dstb[c, pl.ds(t * 16, 16)]
                        plsc.addupdate_scatter(
                            hist_v,
                            [lax.shift_right_logical(idx, 7),
                             lax.bitwise_and(idx, 127)],
                            ones16)
            load_blk(blk + 1)
            pltpu.async_copy(feat.at[srcb.at[0]], rows0, gsem)

        load_blk(0)
        pltpu.async_copy(feat.at[srcb.at[0]], rows0, gsem)
        blk_body(0, True)

        def fori_body(blk, carry):
            blk_body(blk, False)
            return carry

        if nblk > 1:
            lax.fori_loop(1, nblk, fori_body, 0)
        pltpu.make_async_copy(feat.at[srcb.at[0]], rows0, gsem).wait()
        pltpu.make_async_copy(bufs[(B - 1) % 2], acc.at[dstb.at[B - 1]],
                              ssem).wait()

        if with_cnt:
            pltpu.sync_copy(hist_v, cacc.at[riota_v], add=True)
        plsc.subcore_barrier()

        pltpu.sync_copy(acc.at[pl.ds(sid * RPT, RPT)],
                        out.at[pl.ds(cid * AROWS + sid * RPT, RPT)])
        if with_cnt:
            @pl.when(sid < HR // 8)
            def _():
                pltpu.sync_copy(cacc.at[pl.ds(sid * 8, 8)],
                                cnt_out.at[pl.ds(cid * HR + sid * 8, 8)])

    return pl.kernel(body, out_type=tuple(out_type), mesh=mesh,
                     scratch_types=scratch,
                     compiler_params=pltpu.CompilerParams(
                         needs_layout_passes=False))


def _combine_body(p0, p1, c0, c1, h, wl, b, wr, o):
    s = p0[...] + p1[...]
    cnt = c0[...] + c1[...]
    mean = s / jnp.maximum(cnt, 1.0)
    o[...] = jnp.maximum(mean @ wl[...] + b[...] + h[...] @ wr[...], 0.0)


def _combine(p0, p1, c0, c1, h, wl, b, wr):
    R = 1000
    grid = (N_NODES // R,)
    return pl.pallas_call(
        _combine_body,
        grid=grid,
        in_specs=[
            pl.BlockSpec((R, D), lambda i: (i, 0)),
            pl.BlockSpec((R, D), lambda i: (i, 0)),
            pl.BlockSpec((R, 1), lambda i: (i, 0)),
            pl.BlockSpec((R, 1), lambda i: (i, 0)),
            pl.BlockSpec((R, D), lambda i: (i, 0)),
            pl.BlockSpec((D, D), lambda i: (0, 0)),
            pl.BlockSpec((1, D), lambda i: (0, 0)),
            pl.BlockSpec((D, D), lambda i: (0, 0)),
        ],
        out_specs=pl.BlockSpec((R, D), lambda i: (i, 0)),
        out_shape=jax.ShapeDtypeStruct((N_NODES, D), jnp.float32),
    )(p0, p1, c0, c1, h, wl, b, wr)


def kernel(x, edge_index, W_l1, b1, W_r1, W_l2, b2, W_r2):
    e = edge_index.shape[1]
    nch = -(-e // (NW * K * B)) * B
    epad = NW * nch * K
    src = edge_index[0].astype(jnp.int32)
    dst = edge_index[1].astype(jnp.int32)
    pad = epad - e
    srcm = jnp.concatenate([src, jnp.zeros((pad,), jnp.int32)])
    dstm = jnp.concatenate([dst, jnp.full((pad,), N_NODES, jnp.int32)])
    srcm = jnp.pad(srcm.reshape(NW, nch, K), ((0, 0), (0, B), (0, 0)))
    dstm = jnp.pad(dstm.reshape(NW, nch, K), ((0, 0), (0, B), (0, 0)))
    srcm = srcm.reshape(NW * (nch + B), K)
    dstm = dstm.reshape(NW * (nch + B), K)
    zacc = jnp.zeros((RPT, D), jnp.float32)
    riota = jnp.arange(HR, dtype=jnp.int32)

    agg1 = _make_agg(nch, with_cnt=True)
    agg2 = _make_agg(nch, with_cnt=False)

    p, cnt = agg1(x, srcm, dstm, zacc, riota)
    cnt = cnt.reshape(NC, CROWS)
    c0 = cnt[0, :N_NODES, None]
    c1 = cnt[1, :N_NODES, None]
    p0 = p[:N_NODES]
    p1 = p[AROWS:AROWS + N_NODES]
    h1 = _combine(p0, p1, c0, c1, x, W_l1, b1.reshape(1, D), W_r1)
    (p2,) = agg2(h1, srcm, dstm, zacc)
    h2 = _combine(p2[:N_NODES], p2[AROWS:AROWS + N_NODES], c0, c1,
                  h1, W_l2, b2.reshape(1, D), W_r2)
    return h2


# speedup vs baseline: 1.7135x; 1.7135x over previous
import jax
import jax.numpy as jnp
from jax import lax
from jax.experimental import pallas as pl
from jax.experimental.pallas import tpu as pltpu
from jax.experimental.pallas import tpu_sc as plsc

N_NODES = 10000
D = 128
NC = 2
NS = 16
NW = NC * NS
K = 128
AROWS = 10240
RPT = AROWS // NS
HR = AROWS // D


def _make_agg(nch: int, with_cnt: bool):
    mesh = plsc.VectorSubcoreMesh(core_axis_name="c", subcore_axis_name="s")
    out_type = [jax.ShapeDtypeStruct((NC * AROWS, D), jnp.float32)]
    scratch = [
        pltpu.VMEM((K,), jnp.int32),
        pltpu.VMEM((K,), jnp.int32),
        pltpu.VMEM((K, D), jnp.float32),
        pltpu.VMEM_SHARED((AROWS, D), jnp.float32),
        pltpu.SemaphoreType.DMA,
    ]
    if with_cnt:
        out_type.append(jax.ShapeDtypeStruct((NC * HR, D), jnp.float32))
        scratch += [
            pltpu.VMEM((HR, D), jnp.float32),
            pltpu.VMEM((HR,), jnp.int32),
            pltpu.VMEM_SHARED((HR, D), jnp.float32),
        ]

    def body(feat, srcm, dstm, zacc, *rest):
        if with_cnt:
            (riota, out, cnt_out,
             src_v, dst_v, rows_v, acc, sem, hist_v, riota_v, cacc) = rest
        else:
            (out, src_v, dst_v, rows_v, acc, sem) = rest
        cid = lax.axis_index("c")
        sid = lax.axis_index("s")
        wid = cid * NS + sid

        pltpu.sync_copy(zacc, rows_v)
        for i in range(RPT // K):
            pltpu.sync_copy(rows_v, acc.at[pl.ds(sid * RPT + i * K, K)])
        if with_cnt:
            pltpu.sync_copy(zacc.at[pl.ds(0, HR)], hist_v)
            pltpu.sync_copy(riota, riota_v)

            @pl.when(sid < HR // 8)
            def _():
                pltpu.sync_copy(rows_v.at[pl.ds(0, 8)],
                                cacc.at[pl.ds(sid * 8, 8)])

        plsc.subcore_barrier()

        ones16 = jnp.ones((16,), jnp.float32)

        def step(j, carry):
            pltpu.sync_copy(srcm.at[pl.ds((wid * nch + j) * K, K)], src_v)
            pltpu.sync_copy(dstm.at[pl.ds((wid * nch + j) * K, K)], dst_v)
            pltpu.async_copy(feat.at[src_v], rows_v, sem).wait()
            pltpu.sync_copy(rows_v, acc.at[dst_v], add=True)
            if with_cnt:
                for t in range(K // 16):
                    idx = dst_v[pl.ds(t * 16, 16)]
                    plsc.addupdate_scatter(
                        hist_v,
                        [lax.shift_right_logical(idx, 7),
                         lax.bitwise_and(idx, 127)],
                        ones16)
            return carry

        lax.fori_loop(0, nch, step, 0)
        if with_cnt:
            pltpu.sync_copy(hist_v, cacc.at[riota_v], add=True)
        plsc.subcore_barrier()

        for i in range(RPT // K):
            pltpu.sync_copy(acc.at[pl.ds(sid * RPT + i * K, K)], rows_v)
            pltpu.sync_copy(rows_v,
                            out.at[pl.ds(cid * AROWS + sid * RPT + i * K, K)])
        if with_cnt:
            @pl.when(sid < HR // 8)
            def _():
                pltpu.sync_copy(cacc.at[pl.ds(sid * 8, 8)],
                                hist_v.at[pl.ds(0, 8)])
                pltpu.sync_copy(hist_v.at[pl.ds(0, 8)],
                                cnt_out.at[pl.ds(cid * HR + sid * 8, 8)])

    return pl.kernel(body, out_type=tuple(out_type), mesh=mesh,
                     scratch_types=scratch,
                     compiler_params=pltpu.CompilerParams(
                         needs_layout_passes=False))


def _combine_body(p0, p1, c0, c1, h, wl, b, wr, o):
    s = p0[...] + p1[...]
    cnt = c0[...] + c1[...]
    mean = s / jnp.maximum(cnt, 1.0)
    o[...] = jnp.maximum(mean @ wl[...] + b[...] + h[...] @ wr[...], 0.0)


def _combine(p0, p1, c0, c1, h, wl, b, wr):
    R = 1000
    grid = (N_NODES // R,)
    return pl.pallas_call(
        _combine_body,
        grid=grid,
        in_specs=[
            pl.BlockSpec((R, D), lambda i: (i, 0)),
            pl.BlockSpec((R, D), lambda i: (i, 0)),
            pl.BlockSpec((R, 1), lambda i: (i, 0)),
            pl.BlockSpec((R, 1), lambda i: (i, 0)),
            pl.BlockSpec((R, D), lambda i: (i, 0)),
            pl.BlockSpec((D, D), lambda i: (0, 0)),
            pl.BlockSpec((1, D), lambda i: (0, 0)),
            pl.BlockSpec((D, D), lambda i: (0, 0)),
        ],
        out_specs=pl.BlockSpec((R, D), lambda i: (i, 0)),
        out_shape=jax.ShapeDtypeStruct((N_NODES, D), jnp.float32),
    )(p0, p1, c0, c1, h, wl, b, wr)


def kernel(x, edge_index, W_l1, b1, W_r1, W_l2, b2, W_r2):
    e = edge_index.shape[1]
    nch = -(-e // (NW * K))
    epad = NW * nch * K
    src = edge_index[0].astype(jnp.int32)
    dst = edge_index[1].astype(jnp.int32)
    pad = epad - e
    srcm = jnp.concatenate([src, jnp.zeros((pad,), jnp.int32)])
    dstm = jnp.concatenate([dst, jnp.full((pad,), N_NODES, jnp.int32)])
    zacc = jnp.zeros((K, D), jnp.float32)
    riota = jnp.arange(HR, dtype=jnp.int32)

    agg1 = _make_agg(nch, with_cnt=True)
    agg2 = _make_agg(nch, with_cnt=False)

    p, cnt = agg1(x, srcm, dstm, zacc, riota)
    cnt = cnt.reshape(NC, AROWS)
    c0 = cnt[0, :N_NODES, None]
    c1 = cnt[1, :N_NODES, None]
    p0 = p[:N_NODES]
    p1 = p[AROWS:AROWS + N_NODES]
    h1 = _combine(p0, p1, c0, c1, x, W_l1, b1.reshape(1, D), W_r1)
    (p2,) = agg2(h1, srcm, dstm, zacc)
    h2 = _combine(p2[:N_NODES], p2[AROWS:AROWS + N_NODES], c0, c1,
                  h1, W_l2, b2.reshape(1, D), W_r2)
    return h2
